# initial kernel scaffold (unmeasured)
import jax
import jax.numpy as jnp
from jax import lax
from jax.experimental import pallas as pl
from jax.experimental.pallas import tpu as pltpu

N_DEV = 8
B = 8
H = 8
D = 128
BS = 16
NB = 512
P_LOC = 512
T_LOC = P_LOC * BS
NEG = -1e30


def _body(lens_ref, q_ref, k_ref, v_ref, bt_ref, out_ref,
          o_gat, ml_gat, send_sems, recv_sems):
    my = lax.axis_index("i")

    barrier = pltpu.get_barrier_semaphore()
    for off in range(1, N_DEV):
        peer = lax.rem(my + off, N_DEV)
        pl.semaphore_signal(barrier, inc=1, device_id=(peer,),
                            device_id_type=pl.DeviceIdType.MESH)
    pl.semaphore_wait(barrier, N_DEV - 1)

    base = my * P_LOC
    cnt_rows = []
    for b in range(B):
        ln = lens_ref[b]
        bt_row = bt_ref[b, :]
        page_iota = lax.broadcasted_iota(jnp.int32, (P_LOC, NB), 0) + base
        k_iota = lax.broadcasted_iota(jnp.int32, (P_LOC, NB), 1)
        match = (bt_row[None, :] == page_iota) & (k_iota < ln)
        cnt_p = jnp.sum(match.astype(jnp.float32), axis=1, keepdims=True)
        cnt_rows.append(jnp.broadcast_to(cnt_p, (P_LOC, BS)).reshape(1, T_LOC))
    cnt = jnp.concatenate(cnt_rows, axis=0)
    valid = cnt > 0.0

    scale = D ** -0.5
    m_cols, l_cols, o_blocks = [], [], []
    for h in range(H):
        q_h = q_ref[:, h * D:(h + 1) * D]
        k_h = k_ref[:, h * D:(h + 1) * D]
        s = lax.dot_general(q_h, k_h, (((1,), (1,)), ((), ())),
                            preferred_element_type=jnp.float32) * scale
        s = jnp.where(valid, s, NEG)
        m_h = jnp.max(s, axis=1, keepdims=True)
        w = cnt * jnp.exp(s - m_h)
        l_h = jnp.sum(w, axis=1, keepdims=True)
        v_h = v_ref[:, h * D:(h + 1) * D]
        o_h = lax.dot_general(w, v_h, (((1,), (0,)), ((), ())),
                              preferred_element_type=jnp.float32)
        m_cols.append(m_h)
        l_cols.append(l_h)
        o_blocks.append(o_h[:, None, :])
    m_loc = jnp.concatenate(m_cols, axis=1)
    l_loc = jnp.concatenate(l_cols, axis=1)
    o_loc = jnp.concatenate(o_blocks, axis=1)

    o_gat[pl.ds(my, 1)] = o_loc[None]
    ml_gat[pl.ds(my, 1)] = jnp.stack([m_loc, l_loc], axis=0)[None]

    sends = []
    for off in range(1, N_DEV):
        dst = lax.rem(my + off, N_DEV)
        r_o = pltpu.make_async_remote_copy(
            src_ref=o_gat.at[my], dst_ref=o_gat.at[my],
            send_sem=send_sems.at[off, 0], recv_sem=recv_sems.at[my, 0],
            device_id=(dst,), device_id_type=pl.DeviceIdType.MESH)
        r_ml = pltpu.make_async_remote_copy(
            src_ref=ml_gat.at[my], dst_ref=ml_gat.at[my],
            send_sem=send_sems.at[off, 1], recv_sem=recv_sems.at[my, 1],
            device_id=(dst,), device_id_type=pl.DeviceIdType.MESH)
        r_o.start()
        r_ml.start()
        sends.append((r_o, r_ml))

    for off in range(1, N_DEV):
        src = lax.rem(my + off, N_DEV)
        w_o = pltpu.make_async_remote_copy(
            src_ref=o_gat.at[src], dst_ref=o_gat.at[src],
            send_sem=send_sems.at[off, 0], recv_sem=recv_sems.at[src, 0],
            device_id=(src,), device_id_type=pl.DeviceIdType.MESH)
        w_ml = pltpu.make_async_remote_copy(
            src_ref=ml_gat.at[src], dst_ref=ml_gat.at[src],
            send_sem=send_sems.at[off, 1], recv_sem=recv_sems.at[src, 1],
            device_id=(src,), device_id_type=pl.DeviceIdType.MESH)
        w_o.wait_recv()
        w_ml.wait_recv()

    for r_o, r_ml in sends:
        r_o.wait_send()
        r_ml.wait_send()

    m_g = ml_gat[:, 0]
    l_g = ml_gat[:, 1]
    o_g = o_gat[...]
    m_tot = jnp.max(m_g, axis=0)
    sc = jnp.exp(m_g - m_tot[None])
    l_tot = jnp.sum(l_g * sc, axis=0)
    o_tot = jnp.sum(o_g * sc[..., None], axis=0) / l_tot[..., None]
    out_ref[...] = o_tot.reshape(B, H * D)


def kernel(Q, K, V, bt, lens):
    q2 = Q.reshape(B, H * D)
    k2 = K.reshape(T_LOC, H * D)
    v2 = V.reshape(T_LOC, H * D)
    out = pl.pallas_call(
        _body,
        out_shape=jax.ShapeDtypeStruct((B, H * D), jnp.float32),
        in_specs=[
            pl.BlockSpec(memory_space=pltpu.SMEM),
            pl.BlockSpec(memory_space=pltpu.VMEM),
            pl.BlockSpec(memory_space=pltpu.VMEM),
            pl.BlockSpec(memory_space=pltpu.VMEM),
            pl.BlockSpec(memory_space=pltpu.VMEM),
        ],
        out_specs=pl.BlockSpec(memory_space=pltpu.VMEM),
        scratch_shapes=[
            pltpu.VMEM((N_DEV, B, H, D), jnp.float32),
            pltpu.VMEM((N_DEV, 2, B, H), jnp.float32),
            pltpu.SemaphoreType.DMA((N_DEV, 2)),
            pltpu.SemaphoreType.DMA((N_DEV, 2)),
        ],
        compiler_params=pltpu.CompilerParams(collective_id=0),
    )(lens, q2, k2, v2, bt)
    return out.reshape(B, 1, H, D)


# baseline (device time: 93482 ns/iter reference)
import jax
import jax.numpy as jnp
from jax import lax
from jax.experimental import pallas as pl
from jax.experimental.pallas import tpu as pltpu

N_DEV = 8
B = 8
H = 8
D = 128
BS = 16
NB = 512
P_LOC = 512
T_LOC = P_LOC * BS
CT = 2048
N_CHUNK = T_LOC // CT
NEG = -1e30


def _body(lens_ref, q_ref, k_hbm, v_hbm, bt_ref, out_ref,
          k_buf, v_buf, copy_sems, o_gat, ml_gat, send_sems, recv_sems):
    my = lax.axis_index("i")

    barrier = pltpu.get_barrier_semaphore()
    for off in range(1, N_DEV):
        peer = lax.rem(my + off, N_DEV)
        pl.semaphore_signal(barrier, inc=1, device_id=(peer,),
                            device_id_type=pl.DeviceIdType.MESH)
    pl.semaphore_wait(barrier, N_DEV - 1)

    def start_copy(c):
        slot = c % 2
        ck = pltpu.make_async_copy(
            k_hbm.at[pl.ds(c * CT, CT), :], k_buf.at[slot],
            copy_sems.at[slot, 0])
        cv = pltpu.make_async_copy(
            v_hbm.at[pl.ds(c * CT, CT), :], v_buf.at[slot],
            copy_sems.at[slot, 1])
        ck.start()
        cv.start()
        return ck, cv

    copies = [start_copy(0)]

    base = my * P_LOC
    cnt_cols = []
    for b in range(B):
        ln = lens_ref[b]
        bt_row = bt_ref[b:b + 1, :]
        page_iota = lax.broadcasted_iota(jnp.int32, (P_LOC, NB), 0) + base
        k_iota = lax.broadcasted_iota(jnp.int32, (P_LOC, NB), 1)
        match = (bt_row == page_iota) & (k_iota < ln)
        cnt_cols.append(
            jnp.sum(match.astype(jnp.float32), axis=1, keepdims=True))
    cnt_pb = jnp.concatenate(cnt_cols, axis=1)
    cnt = jnp.broadcast_to(
        cnt_pb[:, None, :], (P_LOC, BS, B)).reshape(T_LOC, B)
    valid = cnt > 0.0

    scale = D ** -0.5
    m_parts = [[] for _ in range(H)]
    l_parts = [[] for _ in range(H)]
    o_parts = [[] for _ in range(H)]
    for c in range(N_CHUNK):
        slot = c % 2
        ck, cv = copies[c]
        ck.wait()
        cv.wait()
        if c + 1 < N_CHUNK:
            copies.append(start_copy(c + 1))
        cnt_c = cnt[c * CT:(c + 1) * CT, :]
        valid_c = valid[c * CT:(c + 1) * CT, :]
        for h in range(H):
            q_h = q_ref[:, h * D:(h + 1) * D]
            k_h = k_buf[slot, :, h * D:(h + 1) * D]
            s = lax.dot_general(k_h, q_h, (((1,), (1,)), ((), ())),
                                preferred_element_type=jnp.float32) * scale
            s = jnp.where(valid_c, s, NEG)
            m_c = jnp.max(s, axis=0, keepdims=True)
            w = cnt_c * jnp.exp(s - m_c)
            l_c = jnp.sum(w, axis=0, keepdims=True)
            v_h = v_buf[slot, :, h * D:(h + 1) * D]
            o_c = lax.dot_general(w, v_h, (((0,), (0,)), ((), ())),
                                  preferred_element_type=jnp.float32)
            m_parts[h].append(m_c)
            l_parts[h].append(l_c)
            o_parts[h].append(o_c)

    m_rows, l_rows, o_blocks = [], [], []
    for h in range(H):
        m_st = jnp.concatenate(m_parts[h], axis=0)
        m_h = jnp.max(m_st, axis=0, keepdims=True)
        sc = jnp.exp(m_st - m_h)
        l_h = jnp.sum(jnp.concatenate(l_parts[h], axis=0) * sc,
                      axis=0, keepdims=True)
        o_h = sum(o_parts[h][c] * sc[c:c + 1, :].T
                  for c in range(N_CHUNK))
        m_rows.append(m_h)
        l_rows.append(l_h)
        o_blocks.append(o_h[:, None, :])
    m_loc = jnp.concatenate(m_rows, axis=0).T
    l_loc = jnp.concatenate(l_rows, axis=0).T
    o_loc = jnp.concatenate(o_blocks, axis=1)

    o_gat[pl.ds(my, 1)] = o_loc[None]
    ml_gat[pl.ds(my, 1)] = jnp.stack([m_loc, l_loc], axis=0)[None]

    sends = []
    for off in range(1, N_DEV):
        dst = lax.rem(my + off, N_DEV)
        r_o = pltpu.make_async_remote_copy(
            src_ref=o_gat.at[my], dst_ref=o_gat.at[my],
            send_sem=send_sems.at[off, 0], recv_sem=recv_sems.at[my, 0],
            device_id=(dst,), device_id_type=pl.DeviceIdType.MESH)
        r_ml = pltpu.make_async_remote_copy(
            src_ref=ml_gat.at[my], dst_ref=ml_gat.at[my],
            send_sem=send_sems.at[off, 1], recv_sem=recv_sems.at[my, 1],
            device_id=(dst,), device_id_type=pl.DeviceIdType.MESH)
        r_o.start()
        r_ml.start()
        sends.append((r_o, r_ml))

    for off in range(1, N_DEV):
        src = lax.rem(my + off, N_DEV)
        w_o = pltpu.make_async_remote_copy(
            src_ref=o_gat.at[src], dst_ref=o_gat.at[src],
            send_sem=send_sems.at[off, 0], recv_sem=recv_sems.at[src, 0],
            device_id=(src,), device_id_type=pl.DeviceIdType.MESH)
        w_ml = pltpu.make_async_remote_copy(
            src_ref=ml_gat.at[src], dst_ref=ml_gat.at[src],
            send_sem=send_sems.at[off, 1], recv_sem=recv_sems.at[src, 1],
            device_id=(src,), device_id_type=pl.DeviceIdType.MESH)
        w_o.wait_recv()
        w_ml.wait_recv()

    for r_o, r_ml in sends:
        r_o.wait_send()
        r_ml.wait_send()

    m_g = ml_gat[:, 0]
    l_g = ml_gat[:, 1]
    o_g = o_gat[...]
    m_tot = jnp.max(m_g, axis=0)
    sc = jnp.exp(m_g - m_tot[None])
    l_tot = jnp.sum(l_g * sc, axis=0)
    o_tot = jnp.sum(o_g * sc[..., None], axis=0) / l_tot[..., None]
    out_ref[...] = o_tot.reshape(B, H * D)


def kernel(Q, K, V, bt, lens):
    q2 = Q.reshape(B, H * D)
    k2 = K.reshape(T_LOC, H * D)
    v2 = V.reshape(T_LOC, H * D)
    out = pl.pallas_call(
        _body,
        out_shape=jax.ShapeDtypeStruct((B, H * D), jnp.float32),
        in_specs=[
            pl.BlockSpec(memory_space=pltpu.SMEM),
            pl.BlockSpec(memory_space=pltpu.VMEM),
            pl.BlockSpec(memory_space=pl.ANY),
            pl.BlockSpec(memory_space=pl.ANY),
            pl.BlockSpec(memory_space=pltpu.VMEM),
        ],
        out_specs=pl.BlockSpec(memory_space=pltpu.VMEM),
        scratch_shapes=[
            pltpu.VMEM((2, CT, H * D), jnp.float32),
            pltpu.VMEM((2, CT, H * D), jnp.float32),
            pltpu.SemaphoreType.DMA((2, 2)),
            pltpu.VMEM((N_DEV, B, H, D), jnp.float32),
            pltpu.VMEM((N_DEV, 2, B, H), jnp.float32),
            pltpu.SemaphoreType.DMA((N_DEV, 2)),
            pltpu.SemaphoreType.DMA((N_DEV, 2)),
        ],
        compiler_params=pltpu.CompilerParams(
            collective_id=0, vmem_limit_bytes=56 * 1024 * 1024),
    )(lens, q2, k2, v2, bt)
    return out.reshape(B, 1, H, D)


# device time: 45073 ns/iter; 2.0740x vs baseline; 2.0740x over previous
import jax
import jax.numpy as jnp
from jax import lax
from jax.experimental import pallas as pl
from jax.experimental.pallas import tpu as pltpu

N_DEV = 8
B = 8
H = 8
D = 128
BS = 16
NB = 512
P_LOC = 512
T_LOC = P_LOC * BS
CP = 128
CT = CP * BS
N_CHUNK = P_LOC // CP
NEG = -1e30


def _body(lens_ref, q_ref, k_hbm, v_hbm, bt_ref, out_ref,
          k_buf, v_buf, copy_sems, o_gat, ml_gat, send_sems, recv_sems):
    my = lax.axis_index("i")

    barrier = pltpu.get_barrier_semaphore()
    for off in range(1, N_DEV):
        peer = lax.rem(my + off, N_DEV)
        pl.semaphore_signal(barrier, inc=1, device_id=(peer,),
                            device_id_type=pl.DeviceIdType.MESH)
    pl.semaphore_wait(barrier, N_DEV - 1)

    def start_copy(c):
        slot = c % 2
        copies = []
        for h in range(H):
            copies.append(pltpu.make_async_copy(
                k_hbm.at[pl.ds(c * CP, CP), :, h, :], k_buf.at[slot, h],
                copy_sems.at[slot, 0, h]))
            copies.append(pltpu.make_async_copy(
                v_hbm.at[pl.ds(c * CP, CP), :, h, :], v_buf.at[slot, h],
                copy_sems.at[slot, 1, h]))
        for cp in copies:
            cp.start()
        return copies

    inflight = [start_copy(0)]

    base = my * P_LOC
    cnt_cols = []
    for b in range(B):
        ln = lens_ref[b]
        bt_row = bt_ref[b:b + 1, :]
        page_iota = lax.broadcasted_iota(jnp.int32, (P_LOC, NB), 0) + base
        k_iota = lax.broadcasted_iota(jnp.int32, (P_LOC, NB), 1)
        match = (bt_row == page_iota) & (k_iota < ln)
        cnt_cols.append(
            jnp.sum(match.astype(jnp.float32), axis=1, keepdims=True))
    cnt_pb = jnp.concatenate(cnt_cols, axis=1)
    cnt = jnp.broadcast_to(
        cnt_pb[:, None, :], (P_LOC, BS, B)).reshape(T_LOC, B)
    valid = cnt > 0.0

    scale = D ** -0.5
    m_parts = [[] for _ in range(H)]
    l_parts = [[] for _ in range(H)]
    o_parts = [[] for _ in range(H)]
    for c in range(N_CHUNK):
        slot = c % 2
        for cp in inflight[c]:
            cp.wait()
        if c + 1 < N_CHUNK:
            inflight.append(start_copy(c + 1))
        cnt_c = cnt[c * CT:(c + 1) * CT, :]
        valid_c = valid[c * CT:(c + 1) * CT, :]
        for h in range(H):
            q_h = q_ref[:, 0, h, :]
            k_h = k_buf[slot, h].reshape(CT, D)
            s = lax.dot_general(k_h, q_h, (((1,), (1,)), ((), ())),
                                preferred_element_type=jnp.float32) * scale
            s = jnp.where(valid_c, s, NEG)
            m_c = jnp.max(s, axis=0, keepdims=True)
            w = cnt_c * jnp.exp(s - m_c)
            l_c = jnp.sum(w, axis=0, keepdims=True)
            v_h = v_buf[slot, h].reshape(CT, D)
            o_c = lax.dot_general(w, v_h, (((0,), (0,)), ((), ())),
                                  preferred_element_type=jnp.float32)
            m_parts[h].append(m_c)
            l_parts[h].append(l_c)
            o_parts[h].append(o_c)

    m_rows, l_rows, o_blocks = [], [], []
    for h in range(H):
        m_st = jnp.concatenate(m_parts[h], axis=0)
        m_h = jnp.max(m_st, axis=0, keepdims=True)
        sc = jnp.exp(m_st - m_h)
        l_h = jnp.sum(jnp.concatenate(l_parts[h], axis=0) * sc,
                      axis=0, keepdims=True)
        o_h = sum(o_parts[h][c] * sc[c:c + 1, :].T
                  for c in range(N_CHUNK))
        m_rows.append(m_h)
        l_rows.append(l_h)
        o_blocks.append(o_h[:, None, :])
    m_loc = jnp.concatenate(m_rows, axis=0).T
    l_loc = jnp.concatenate(l_rows, axis=0).T
    o_loc = jnp.concatenate(o_blocks, axis=1)

    o_gat[pl.ds(my, 1)] = o_loc[None]
    ml_gat[pl.ds(my, 1)] = jnp.stack([m_loc, l_loc], axis=0)[None]

    sends = []
    for off in range(1, N_DEV):
        dst = lax.rem(my + off, N_DEV)
        r_o = pltpu.make_async_remote_copy(
            src_ref=o_gat.at[my], dst_ref=o_gat.at[my],
            send_sem=send_sems.at[off, 0], recv_sem=recv_sems.at[my, 0],
            device_id=(dst,), device_id_type=pl.DeviceIdType.MESH)
        r_ml = pltpu.make_async_remote_copy(
            src_ref=ml_gat.at[my], dst_ref=ml_gat.at[my],
            send_sem=send_sems.at[off, 1], recv_sem=recv_sems.at[my, 1],
            device_id=(dst,), device_id_type=pl.DeviceIdType.MESH)
        r_o.start()
        r_ml.start()
        sends.append((r_o, r_ml))

    for off in range(1, N_DEV):
        src = lax.rem(my + off, N_DEV)
        w_o = pltpu.make_async_remote_copy(
            src_ref=o_gat.at[src], dst_ref=o_gat.at[src],
            send_sem=send_sems.at[off, 0], recv_sem=recv_sems.at[src, 0],
            device_id=(src,), device_id_type=pl.DeviceIdType.MESH)
        w_ml = pltpu.make_async_remote_copy(
            src_ref=ml_gat.at[src], dst_ref=ml_gat.at[src],
            send_sem=send_sems.at[off, 1], recv_sem=recv_sems.at[src, 1],
            device_id=(src,), device_id_type=pl.DeviceIdType.MESH)
        w_o.wait_recv()
        w_ml.wait_recv()

    for r_o, r_ml in sends:
        r_o.wait_send()
        r_ml.wait_send()

    m_g = ml_gat[:, 0]
    l_g = ml_gat[:, 1]
    o_g = o_gat[...]
    m_tot = jnp.max(m_g, axis=0)
    sc = jnp.exp(m_g - m_tot[None])
    l_tot = jnp.sum(l_g * sc, axis=0)
    o_tot = jnp.sum(o_g * sc[..., None], axis=0) / l_tot[..., None]
    out_ref[:, 0, :, :] = o_tot


def kernel(Q, K, V, bt, lens):
    return pl.pallas_call(
        _body,
        out_shape=jax.ShapeDtypeStruct((B, 1, H, D), jnp.float32),
        in_specs=[
            pl.BlockSpec(memory_space=pltpu.SMEM),
            pl.BlockSpec(memory_space=pltpu.VMEM),
            pl.BlockSpec(memory_space=pl.ANY),
            pl.BlockSpec(memory_space=pl.ANY),
            pl.BlockSpec(memory_space=pltpu.VMEM),
        ],
        out_specs=pl.BlockSpec(memory_space=pltpu.VMEM),
        scratch_shapes=[
            pltpu.VMEM((2, H, CP, BS, D), jnp.float32),
            pltpu.VMEM((2, H, CP, BS, D), jnp.float32),
            pltpu.SemaphoreType.DMA((2, 2, H)),
            pltpu.VMEM((N_DEV, B, H, D), jnp.float32),
            pltpu.VMEM((N_DEV, 2, B, H), jnp.float32),
            pltpu.SemaphoreType.DMA((N_DEV, 2)),
            pltpu.SemaphoreType.DMA((N_DEV, 2)),
        ],
        compiler_params=pltpu.CompilerParams(
            collective_id=0, vmem_limit_bytes=56 * 1024 * 1024),
    )(lens, Q, K, V, bt)


# device time: 37733 ns/iter; 2.4775x vs baseline; 1.1945x over previous
import jax
import jax.numpy as jnp
from jax import lax
from jax.experimental import pallas as pl
from jax.experimental.pallas import tpu as pltpu

N_DEV = 8
B = 8
H = 8
D = 128
BS = 16
NB = 512
P_LOC = 512
T_LOC = P_LOC * BS
CP = 128
CT = CP * BS
N_CHUNK = P_LOC // CP
NEG = -1e30


def _body(lens_ref, q_ref, k_hbm, v_hbm, bt_ref, out_ref,
          k_buf, v_buf, copy_sems, o_gat, ml_gat, send_sems, recv_sems):
    my = lax.axis_index("i")

    def start_copy(c):
        slot = c % 2
        copies = []
        for h in range(H):
            copies.append(pltpu.make_async_copy(
                k_hbm.at[pl.ds(c * CP, CP), :, h, :], k_buf.at[slot, h],
                copy_sems.at[slot, 0, h]))
            copies.append(pltpu.make_async_copy(
                v_hbm.at[pl.ds(c * CP, CP), :, h, :], v_buf.at[slot, h],
                copy_sems.at[slot, 1, h]))
        for cp in copies:
            cp.start()
        return copies

    inflight = [start_copy(0)]

    barrier = pltpu.get_barrier_semaphore()
    for off in range(1, N_DEV):
        peer = lax.rem(my + off, N_DEV)
        pl.semaphore_signal(barrier, inc=1, device_id=(peer,),
                            device_id_type=pl.DeviceIdType.MESH)
    pl.semaphore_wait(barrier, N_DEV - 1)

    base = my * P_LOC
    k_iota_row = lax.broadcasted_iota(jnp.int32, (1, NB), 1)
    cnt_cols = []
    for b in range(B):
        ln = lens_ref[b]
        bt_row = jnp.where(k_iota_row < ln, bt_ref[b:b + 1, :], -1)
        page_iota = lax.broadcasted_iota(jnp.int32, (P_LOC, NB), 0) + base
        match = bt_row == page_iota
        cnt_cols.append(
            jnp.sum(match.astype(jnp.float32), axis=1, keepdims=True))
    cnt_bp = jnp.concatenate(cnt_cols, axis=1).T

    e_row = lax.broadcasted_iota(jnp.int32, (CP, CT), 0)
    e_col = lax.broadcasted_iota(jnp.int32, (CP, CT), 1) // BS
    expand = (e_row == e_col).astype(jnp.float32)

    scale = D ** -0.5
    m_parts = [[] for _ in range(H)]
    l_parts = [[] for _ in range(H)]
    o_parts = [[] for _ in range(H)]
    for c in range(N_CHUNK):
        slot = c % 2
        for cp in inflight[c]:
            cp.wait()
        if c + 1 < N_CHUNK:
            inflight.append(start_copy(c + 1))
        cnt_c = lax.dot_general(
            cnt_bp[:, c * CP:(c + 1) * CP], expand,
            (((1,), (0,)), ((), ())),
            preferred_element_type=jnp.float32)
        for h in range(H):
            q_h = q_ref[:, 0, h, :]
            k_h = k_buf[slot, h].reshape(CT, D)
            s = lax.dot_general(q_h, k_h, (((1,), (1,)), ((), ())),
                                preferred_element_type=jnp.float32) * scale
            m_c = jnp.max(s, axis=1, keepdims=True)
            w = cnt_c * jnp.exp(s - m_c)
            l_c = jnp.sum(w, axis=1, keepdims=True)
            v_h = v_buf[slot, h].reshape(CT, D)
            o_c = lax.dot_general(w, v_h, (((1,), (0,)), ((), ())),
                                  preferred_element_type=jnp.float32)
            m_parts[h].append(m_c)
            l_parts[h].append(l_c)
            o_parts[h].append(o_c)

    m_rows, l_rows, o_blocks = [], [], []
    for h in range(H):
        m_st = jnp.concatenate(m_parts[h], axis=1)
        m_h = jnp.max(m_st, axis=1, keepdims=True)
        sc = jnp.exp(m_st - m_h)
        l_h = jnp.sum(jnp.concatenate(l_parts[h], axis=1) * sc,
                      axis=1, keepdims=True)
        o_h = sum(o_parts[h][c] * sc[:, c:c + 1]
                  for c in range(N_CHUNK))
        m_rows.append(m_h)
        l_rows.append(l_h)
        o_blocks.append(o_h[:, None, :])
    m_loc = jnp.concatenate(m_rows, axis=1)
    l_loc = jnp.concatenate(l_rows, axis=1)
    o_loc = jnp.concatenate(o_blocks, axis=1)

    o_gat[pl.ds(my, 1)] = o_loc[None]
    ml_gat[pl.ds(my, 1)] = jnp.stack([m_loc, l_loc], axis=0)[None]

    sends = []
    for off in range(1, N_DEV):
        dst = lax.rem(my + off, N_DEV)
        r_o = pltpu.make_async_remote_copy(
            src_ref=o_gat.at[my], dst_ref=o_gat.at[my],
            send_sem=send_sems.at[off, 0], recv_sem=recv_sems.at[my, 0],
            device_id=(dst,), device_id_type=pl.DeviceIdType.MESH)
        r_ml = pltpu.make_async_remote_copy(
            src_ref=ml_gat.at[my], dst_ref=ml_gat.at[my],
            send_sem=send_sems.at[off, 1], recv_sem=recv_sems.at[my, 1],
            device_id=(dst,), device_id_type=pl.DeviceIdType.MESH)
        r_o.start()
        r_ml.start()
        sends.append((r_o, r_ml))

    for off in range(1, N_DEV):
        src = lax.rem(my + off, N_DEV)
        w_o = pltpu.make_async_remote_copy(
            src_ref=o_gat.at[src], dst_ref=o_gat.at[src],
            send_sem=send_sems.at[off, 0], recv_sem=recv_sems.at[src, 0],
            device_id=(src,), device_id_type=pl.DeviceIdType.MESH)
        w_ml = pltpu.make_async_remote_copy(
            src_ref=ml_gat.at[src], dst_ref=ml_gat.at[src],
            send_sem=send_sems.at[off, 1], recv_sem=recv_sems.at[src, 1],
            device_id=(src,), device_id_type=pl.DeviceIdType.MESH)
        w_o.wait_recv()
        w_ml.wait_recv()

    for r_o, r_ml in sends:
        r_o.wait_send()
        r_ml.wait_send()

    m_g = ml_gat[:, 0]
    l_g = ml_gat[:, 1]
    o_g = o_gat[...]
    m_tot = jnp.max(m_g, axis=0)
    sc = jnp.exp(m_g - m_tot[None])
    l_tot = jnp.sum(l_g * sc, axis=0)
    o_tot = jnp.sum(o_g * sc[..., None], axis=0) / l_tot[..., None]
    out_ref[:, 0, :, :] = o_tot


def kernel(Q, K, V, bt, lens):
    return pl.pallas_call(
        _body,
        out_shape=jax.ShapeDtypeStruct((B, 1, H, D), jnp.float32),
        in_specs=[
            pl.BlockSpec(memory_space=pltpu.SMEM),
            pl.BlockSpec(memory_space=pltpu.VMEM),
            pl.BlockSpec(memory_space=pl.ANY),
            pl.BlockSpec(memory_space=pl.ANY),
            pl.BlockSpec(memory_space=pltpu.VMEM),
        ],
        out_specs=pl.BlockSpec(memory_space=pltpu.VMEM),
        scratch_shapes=[
            pltpu.VMEM((2, H, CP, BS, D), jnp.float32),
            pltpu.VMEM((2, H, CP, BS, D), jnp.float32),
            pltpu.SemaphoreType.DMA((2, 2, H)),
            pltpu.VMEM((N_DEV, B, H, D), jnp.float32),
            pltpu.VMEM((N_DEV, 2, B, H), jnp.float32),
            pltpu.SemaphoreType.DMA((N_DEV, 2)),
            pltpu.SemaphoreType.DMA((N_DEV, 2)),
        ],
        compiler_params=pltpu.CompilerParams(
            collective_id=0, vmem_limit_bytes=56 * 1024 * 1024),
    )(lens, Q, K, V, bt)
